# Initial kernel scaffold; baseline (speedup 1.0000x reference)
import jax
import jax.numpy as jnp
from jax.experimental import pallas as pl


def kernel(sample_feature, sample_xyz, skip_feature, skip_xyz, W1, b1, g1, be1, W2, b2, g2, be2):
    raise NotImplementedError("write your pallas kernel here")


# speedup vs baseline: 23.5483x; 23.5483x over previous
import functools

import jax
import jax.numpy as jnp
from jax.experimental import pallas as pl
from jax.experimental.pallas import tpu as pltpu

B, S, N = 8, 1024, 4096
DIN, DOUT = 512, 256
EPS = 1e-5
M1 = B * S
M2 = B * N
RB = 512
NBLK = N // RB


def _mm_bn_relu_small(x_ref, w_ref, b_ref, g_ref, be_ref, o_ref):
    y = jnp.dot(x_ref[...], w_ref[...], preferred_element_type=jnp.float32)
    y = y + b_ref[...]
    mu = jnp.mean(y, axis=0, keepdims=True)
    d = y - mu
    var = jnp.mean(d * d, axis=0, keepdims=True)
    o_ref[...] = jnp.maximum(g_ref[...] * d * jax.lax.rsqrt(var + EPS) + be_ref[...], 0.0)


def _knn_body(skip_ref, samp_ref, idx_ref, w_ref):
    sk = skip_ref[0]
    sp = samp_ref[0]
    dx = sk[:, 0:1] - sp[0:1, :]
    dy = sk[:, 1:2] - sp[1:2, :]
    dz = sk[:, 2:3] - sp[2:3, :]
    d2 = (dx * dx + dy * dy) + dz * dz
    lane = jax.lax.broadcasted_iota(jnp.int32, (RB, S), 1)
    d = d2
    vals, idxs = [], []
    for k in range(3):
        mk = jnp.min(d, axis=1, keepdims=True)
        ik = jnp.min(jnp.where(d == mk, lane, S), axis=1, keepdims=True)
        vals.append(mk)
        idxs.append(ik)
        if k < 2:
            d = jnp.where(lane == ik, jnp.float32(jnp.inf), d)
    dists = jnp.concatenate(vals, axis=1)
    recip = 1.0 / (dists + 1e-8)
    w = recip / jnp.sum(recip, axis=1, keepdims=True)
    idx_ref[...] = jnp.concatenate(idxs, axis=1)
    w_ref[...] = w


def _mm2_stats_body(x_ref, w_ref, b_ref, y_ref, s_ref, ss_ref):
    i = pl.program_id(0)
    y = jnp.dot(x_ref[...], w_ref[...], preferred_element_type=jnp.float32)
    y = y + b_ref[...]
    y_ref[...] = y

    @pl.when(i == 0)
    def _():
        s_ref[...] = jnp.zeros_like(s_ref)
        ss_ref[...] = jnp.zeros_like(ss_ref)

    s_ref[...] += jnp.sum(y, axis=0, keepdims=True)
    ss_ref[...] += jnp.sum(y * y, axis=0, keepdims=True)


def _interp_add_body(y_ref, s_ref, ss_ref, g_ref, be_ref, sf_ref, idx_ref, w_ref, o_ref):
    mu = s_ref[...] / M2
    var = ss_ref[...] / M2 - mu * mu
    sk = jnp.maximum(g_ref[...] * (y_ref[...] - mu) * jax.lax.rsqrt(var + EPS) + be_ref[...], 0.0)
    iw = idx_ref[...]
    ww = w_ref[...]
    lane = jax.lax.broadcasted_iota(jnp.int32, (RB, S), 1)
    p = jnp.where(lane == iw[:, 0:1], ww[:, 0:1], 0.0)
    p = p + jnp.where(lane == iw[:, 1:2], ww[:, 1:2], 0.0)
    p = p + jnp.where(lane == iw[:, 2:3], ww[:, 2:3], 0.0)
    interp = jnp.dot(p, sf_ref[...], preferred_element_type=jnp.float32)
    o_ref[...] = interp + sk


def kernel(sample_feature, sample_xyz, skip_feature, skip_xyz,
           W1, b1, g1, be1, W2, b2, g2, be2):
    f32 = jnp.float32
    x1 = sample_feature.reshape(M1, DIN)
    x2 = skip_feature.reshape(M2, DOUT)
    w1t = W1.T
    w2t = W2.T
    samp_t = sample_xyz.transpose(0, 2, 1)
    row = lambda v: v.reshape(1, DOUT)

    sf = pl.pallas_call(
        _mm_bn_relu_small,
        out_shape=jax.ShapeDtypeStruct((M1, DOUT), f32),
    )(x1, w1t, row(b1), row(g1), row(be1))

    idx, w = pl.pallas_call(
        _knn_body,
        grid=(B, NBLK),
        in_specs=[
            pl.BlockSpec((1, RB, 3), lambda b, j: (b, j, 0)),
            pl.BlockSpec((1, 3, S), lambda b, j: (b, 0, 0)),
        ],
        out_specs=[
            pl.BlockSpec((RB, 3), lambda b, j: (b * NBLK + j, 0)),
            pl.BlockSpec((RB, 3), lambda b, j: (b * NBLK + j, 0)),
        ],
        out_shape=[
            jax.ShapeDtypeStruct((M2, 3), jnp.int32),
            jax.ShapeDtypeStruct((M2, 3), f32),
        ],
    )(skip_xyz, samp_t)

    y2, s2, ss2 = pl.pallas_call(
        _mm2_stats_body,
        grid=(M2 // 1024,),
        in_specs=[
            pl.BlockSpec((1024, DOUT), lambda i: (i, 0)),
            pl.BlockSpec((DOUT, DOUT), lambda i: (0, 0)),
            pl.BlockSpec((1, DOUT), lambda i: (0, 0)),
        ],
        out_specs=[
            pl.BlockSpec((1024, DOUT), lambda i: (i, 0)),
            pl.BlockSpec((1, DOUT), lambda i: (0, 0)),
            pl.BlockSpec((1, DOUT), lambda i: (0, 0)),
        ],
        out_shape=[
            jax.ShapeDtypeStruct((M2, DOUT), f32),
            jax.ShapeDtypeStruct((1, DOUT), f32),
            jax.ShapeDtypeStruct((1, DOUT), f32),
        ],
    )(x2, w2t, row(b2))

    out2d = pl.pallas_call(
        _interp_add_body,
        grid=(B, NBLK),
        in_specs=[
            pl.BlockSpec((RB, DOUT), lambda b, j: (b * NBLK + j, 0)),
            pl.BlockSpec((1, DOUT), lambda b, j: (0, 0)),
            pl.BlockSpec((1, DOUT), lambda b, j: (0, 0)),
            pl.BlockSpec((1, DOUT), lambda b, j: (0, 0)),
            pl.BlockSpec((1, DOUT), lambda b, j: (0, 0)),
            pl.BlockSpec((S, DOUT), lambda b, j: (b, 0)),
            pl.BlockSpec((RB, 3), lambda b, j: (b * NBLK + j, 0)),
            pl.BlockSpec((RB, 3), lambda b, j: (b * NBLK + j, 0)),
        ],
        out_specs=pl.BlockSpec((RB, DOUT), lambda b, j: (b * NBLK + j, 0)),
        out_shape=jax.ShapeDtypeStruct((M2, DOUT), f32),
    )(y2, s2, ss2, row(g2), row(be2), sf, idx, w)

    return (out2d.reshape(B, N, DOUT), skip_xyz)
